# baseline (speedup 1.0000x reference)
import functools

import jax
import jax.numpy as jnp
from jax import lax
from jax.experimental import pallas as pl
from jax.experimental.pallas import tpu as pltpu


def _fused_grud_kernel(inp_ref, gx_ref, wbig_ref, bbig_ref, wzr_ref, whh_ref,
                       out_ref, gates_scr, dh_scr, h_scr, *, hidden_p, block_t,
                       unroll):
    Hp = hidden_p

    @pl.when(pl.program_id(1) == 0)
    def _():
        h_scr[...] = jnp.zeros_like(h_scr)

    x = inp_ref[:, 0]
    x_last = inp_ref[:, 1]
    mask = inp_ref[:, 2]
    delta = inp_ref[:, 3]
    diag = gx_ref[0:1, :]
    gbias = gx_ref[1:2, :]
    decay_x = jnp.exp(-jnp.maximum(0.0, delta * diag + gbias))
    x_hat = mask * x + (1.0 - mask) * (decay_x * x_last)
    rows = jnp.concatenate([x_hat, mask, delta], axis=-1)
    tc, bc = rows.shape[0], rows.shape[1]
    rows = rows.reshape(tc * bc, rows.shape[2]).astype(jnp.bfloat16)
    pre = jnp.dot(rows, wbig_ref[...], preferred_element_type=jnp.float32)
    pre = pre + bbig_ref[0:1, :]
    pre = pre.reshape(tc, bc, 4 * Hp)
    gates_scr[...] = pre[..., :3 * Hp]
    dh_scr[...] = jnp.exp(-jnp.maximum(0.0, pre[..., 3 * Hp:]))

    def sigmoid_t(x):
        return 0.5 * jnp.tanh(0.5 * x) + 0.5

    bc = h_scr.shape[0]
    nchain = 2
    cw = bc // nchain

    def step(t, _):
        d = dh_scr[t]
        g = gates_scr[t]
        a = [None] * nchain
        zr = [None] * nchain
        for i in range(nchain):
            s = slice(i * cw, (i + 1) * cw)
            a[i] = d[s] * h_scr[s]
            zr[i] = jnp.dot(a[i].astype(jnp.bfloat16), wzr_ref[...],
                            preferred_element_type=jnp.float32)
        for i in range(nchain):
            s = slice(i * cw, (i + 1) * cw)
            z = sigmoid_t(g[s, :Hp] + zr[i][:, :Hp])
            r = sigmoid_t(g[s, Hp:2 * Hp] + zr[i][:, Hp:])
            hh = jnp.dot((r * a[i]).astype(jnp.bfloat16), whh_ref[...],
                         preferred_element_type=jnp.float32)
            h_tilde = jnp.tanh(g[s, 2 * Hp:] + hh)
            h = (1.0 - z) * a[i] + z * h_tilde
            h_scr[s] = h
            out_ref[t, s] = h
        return 0

    lax.fori_loop(0, block_t, step, 0, unroll=unroll)


def kernel(inp, gx_diag, gx_bias, wgh, bgh, wzx, wzh, wzm, bz,
           wrx, wrh, wrm, br, whx, whh, whm, bh):
    B, C, T, D = inp.shape
    assert C == 4
    H = wzh.shape[0]
    assert H % 128 == 0 and D % 128 == 0 and B % 16 == 0
    Hp = H

    zeros_dh = jnp.zeros((D, H), jnp.float32)
    w_big = jnp.concatenate([
        jnp.concatenate([wzx, wzm, zeros_dh], axis=0),
        jnp.concatenate([wrx, wrm, zeros_dh], axis=0),
        jnp.concatenate([whx, whm, zeros_dh], axis=0),
        jnp.concatenate([zeros_dh, zeros_dh, wgh], axis=0),
    ], axis=1).astype(jnp.bfloat16)
    b_big = jnp.concatenate([bz, br, bh, bgh], axis=1)
    b_big = jnp.pad(b_big, ((0, 7), (0, 0)))
    gx = jnp.pad(jnp.concatenate([gx_diag, gx_bias], axis=0),
                 ((0, 6), (0, 0)))

    w_zr = jnp.concatenate([wzh, wrh], axis=1).astype(jnp.bfloat16)
    w_hh = whh.astype(jnp.bfloat16)

    inp_t = jnp.transpose(inp, (2, 1, 0, 3))

    nb = 1
    block_b = B // nb
    block_t = 64 if T % 64 == 0 else [t for t in range(1, T + 1)
                                      if T % t == 0 and t % 8 == 0][-1]
    nt = T // block_t
    unroll = 2

    kernel_fn = functools.partial(_fused_grud_kernel, hidden_p=Hp,
                                  block_t=block_t, unroll=unroll)

    out = pl.pallas_call(
        kernel_fn,
        out_shape=jax.ShapeDtypeStruct((T, B, Hp), jnp.float32),
        grid=(nb, nt),
        in_specs=[
            pl.BlockSpec((block_t, 4, block_b, D), lambda b, c: (c, 0, b, 0)),
            pl.BlockSpec((8, D), lambda b, c: (0, 0)),
            pl.BlockSpec((3 * D, 4 * Hp), lambda b, c: (0, 0)),
            pl.BlockSpec((8, 4 * Hp), lambda b, c: (0, 0)),
            pl.BlockSpec((Hp, 2 * Hp), lambda b, c: (0, 0)),
            pl.BlockSpec((Hp, Hp), lambda b, c: (0, 0)),
        ],
        out_specs=pl.BlockSpec((block_t, block_b, Hp),
                               lambda b, c: (c, b, 0)),
        scratch_shapes=[
            pltpu.VMEM((block_t, block_b, 3 * Hp), jnp.float32),
            pltpu.VMEM((block_t, block_b, Hp), jnp.float32),
            pltpu.VMEM((block_b, Hp), jnp.float32),
        ],
        compiler_params=pltpu.CompilerParams(
            dimension_semantics=("parallel", "arbitrary"),
            vmem_limit_bytes=64 * 1024 * 1024,
        ),
    )(inp_t, gx, w_big, b_big, w_zr, w_hh)

    return jnp.transpose(out[..., :H], (1, 0, 2))


# speedup vs baseline: 1.3021x; 1.3021x over previous
import functools

import jax
import jax.numpy as jnp
from jax import lax
from jax.experimental import pallas as pl
from jax.experimental.pallas import tpu as pltpu


def _fused_grud_kernel(inp_ref, gx_ref, wbig_ref, bbig_ref, wzr_ref, whh_ref,
                       out_ref, gates_scr, dh_scr, h_scr, stage_scr, *,
                       hidden_p, block_t, unroll):
    Hp = hidden_p

    @pl.when(pl.program_id(1) == 0)
    def _():
        h_scr[...] = jnp.zeros_like(h_scr)

    x = inp_ref[:, 0]
    x_last = inp_ref[:, 1]
    mask = inp_ref[:, 2]
    delta = inp_ref[:, 3]
    diag = gx_ref[0:1, :]
    gbias = gx_ref[1:2, :]
    decay_x = jnp.exp(-jnp.maximum(0.0, delta * diag + gbias))
    x_hat = mask * x + (1.0 - mask) * (decay_x * x_last)
    rows = jnp.concatenate([x_hat, mask, delta], axis=-1)
    rows = jnp.swapaxes(rows, 0, 1)
    tc, bc = rows.shape[0], rows.shape[1]
    rows = rows.reshape(tc * bc, rows.shape[2]).astype(jnp.bfloat16)
    pre = jnp.dot(rows, wbig_ref[...], preferred_element_type=jnp.float32)
    pre = pre + bbig_ref[0:1, :]
    pre = pre.reshape(tc, bc, 4 * Hp)
    gates_scr[...] = pre[..., :3 * Hp]
    dh_scr[...] = jnp.exp(-jnp.maximum(0.0, pre[..., 3 * Hp:]))

    def sigmoid_t(x):
        return 0.5 * jnp.tanh(0.5 * x) + 0.5

    def step(t, _):
        a = dh_scr[t] * h_scr[...]
        zr = jnp.dot(a.astype(jnp.bfloat16), wzr_ref[...],
                     preferred_element_type=jnp.float32)
        g = gates_scr[t]
        z = sigmoid_t(g[:, :Hp] + zr[:, :Hp])
        r = sigmoid_t(g[:, Hp:2 * Hp] + zr[:, Hp:])
        hh = jnp.dot((r * a).astype(jnp.bfloat16), whh_ref[...],
                     preferred_element_type=jnp.float32)
        h_tilde = jnp.tanh(g[:, 2 * Hp:] + hh)
        h = (1.0 - z) * a + z * h_tilde
        h_scr[...] = h
        stage_scr[t] = h
        return 0

    lax.fori_loop(0, block_t, step, 0, unroll=unroll)

    out_ref[...] = jnp.swapaxes(stage_scr[...], 0, 1)


def kernel(inp, gx_diag, gx_bias, wgh, bgh, wzx, wzh, wzm, bz,
           wrx, wrh, wrm, br, whx, whh, whm, bh):
    B, C, T, D = inp.shape
    assert C == 4
    H = wzh.shape[0]
    assert H % 128 == 0 and D % 128 == 0 and B % 16 == 0
    Hp = H

    zeros_dh = jnp.zeros((D, H), jnp.float32)
    w_big = jnp.concatenate([
        jnp.concatenate([wzx, wzm, zeros_dh], axis=0),
        jnp.concatenate([wrx, wrm, zeros_dh], axis=0),
        jnp.concatenate([whx, whm, zeros_dh], axis=0),
        jnp.concatenate([zeros_dh, zeros_dh, wgh], axis=0),
    ], axis=1).astype(jnp.bfloat16)
    b_big = jnp.concatenate([bz, br, bh, bgh], axis=1)
    b_big = jnp.pad(b_big, ((0, 7), (0, 0)))
    gx = jnp.pad(jnp.concatenate([gx_diag, gx_bias], axis=0),
                 ((0, 6), (0, 0)))

    w_zr = jnp.concatenate([wzh, wrh], axis=1).astype(jnp.bfloat16)
    w_hh = whh.astype(jnp.bfloat16)

    nb = 1
    block_b = B // nb
    block_t = 64 if T % 64 == 0 else [t for t in range(1, T + 1)
                                      if T % t == 0 and t % 8 == 0][-1]
    nt = T // block_t
    unroll = 2

    kernel_fn = functools.partial(_fused_grud_kernel, hidden_p=Hp,
                                  block_t=block_t, unroll=unroll)

    out = pl.pallas_call(
        kernel_fn,
        out_shape=jax.ShapeDtypeStruct((B, T, Hp), jnp.float32),
        grid=(nb, nt),
        in_specs=[
            pl.BlockSpec((block_b, 4, block_t, D), lambda b, c: (b, 0, c, 0)),
            pl.BlockSpec((8, D), lambda b, c: (0, 0)),
            pl.BlockSpec((3 * D, 4 * Hp), lambda b, c: (0, 0)),
            pl.BlockSpec((8, 4 * Hp), lambda b, c: (0, 0)),
            pl.BlockSpec((Hp, 2 * Hp), lambda b, c: (0, 0)),
            pl.BlockSpec((Hp, Hp), lambda b, c: (0, 0)),
        ],
        out_specs=pl.BlockSpec((block_b, block_t, Hp),
                               lambda b, c: (b, c, 0)),
        scratch_shapes=[
            pltpu.VMEM((block_t, block_b, 3 * Hp), jnp.float32),
            pltpu.VMEM((block_t, block_b, Hp), jnp.float32),
            pltpu.VMEM((block_b, Hp), jnp.float32),
            pltpu.VMEM((block_t, block_b, Hp), jnp.float32),
        ],
        compiler_params=pltpu.CompilerParams(
            dimension_semantics=("arbitrary", "arbitrary"),
            vmem_limit_bytes=64 * 1024 * 1024,
        ),
    )(inp, gx, w_big, b_big, w_zr, w_hh)

    return out[..., :H]
